# baseline (speedup 1.0000x reference)
import jax
import jax.numpy as jnp
from jax.experimental import pallas as pl
from jax.experimental.pallas import tpu as pltpu


def _copy_body(u_in, i_in, u_out, i_out, sem_u, sem_i):
    cu = pltpu.make_async_copy(u_in, u_out, sem_u)
    ci = pltpu.make_async_copy(i_in, i_out, sem_i)
    cu.start()
    ci.start()
    cu.wait()
    ci.wait()


def kernel(user_weight, item_weight, edge_index):
    out_shape = (
        jax.ShapeDtypeStruct(user_weight.shape, user_weight.dtype),
        jax.ShapeDtypeStruct(item_weight.shape, item_weight.dtype),
    )
    return pl.pallas_call(
        _copy_body,
        in_specs=[
            pl.BlockSpec(memory_space=pl.ANY),
            pl.BlockSpec(memory_space=pl.ANY),
        ],
        out_specs=(
            pl.BlockSpec(memory_space=pl.ANY),
            pl.BlockSpec(memory_space=pl.ANY),
        ),
        out_shape=out_shape,
        scratch_shapes=[pltpu.SemaphoreType.DMA, pltpu.SemaphoreType.DMA],
    )(user_weight, item_weight)


# speedup vs baseline: 17.9557x; 17.9557x over previous
import jax
import jax.numpy as jnp
from jax.experimental import pallas as pl
from jax.experimental.pallas import tpu as pltpu

_GRID = 100


def _copy_body(u_in, i_in, u_out, i_out):
    u_out[...] = u_in[...]
    i_out[...] = i_in[...]


def kernel(user_weight, item_weight, edge_index):
    nu, d = user_weight.shape
    ni, _ = item_weight.shape
    bu, bi = nu // _GRID, ni // _GRID
    out_shape = (
        jax.ShapeDtypeStruct(user_weight.shape, user_weight.dtype),
        jax.ShapeDtypeStruct(item_weight.shape, item_weight.dtype),
    )
    return pl.pallas_call(
        _copy_body,
        grid=(_GRID,),
        in_specs=[
            pl.BlockSpec((bu, d), lambda g: (g, 0)),
            pl.BlockSpec((bi, d), lambda g: (g, 0)),
        ],
        out_specs=(
            pl.BlockSpec((bu, d), lambda g: (g, 0)),
            pl.BlockSpec((bi, d), lambda g: (g, 0)),
        ),
        out_shape=out_shape,
        compiler_params=pltpu.CompilerParams(
            dimension_semantics=("arbitrary",),
        ),
    )(user_weight, item_weight)
